# baseline (device time: 994414 ns/iter reference)
import os

import jax

os.makedirs("/tmp/scband_jax_cache", exist_ok=True)
jax.config.update("jax_compilation_cache_dir", "/tmp/scband_jax_cache")
jax.config.update("jax_persistent_cache_min_compile_time_secs", 0)

import jax.numpy as jnp
from jax import lax
from jax.experimental import pallas as pl
from jax.experimental.pallas import tpu as pltpu

N_DEV = 4


def kernel(A, B):
    m_per, k = A.shape
    _, n = B.shape
    M = N_DEV * m_per
    half = m_per // 2

    TM = 512
    n_tiles = m_per // TM
    n_ctiles = n_tiles // 2
    n_atiles = n_tiles - n_ctiles
    piece = half // 2

    def body(a_ref, b_ref, out_ref, ag_ref, a_vmem, c_vmem, local_sem,
             sAR, rAR, sAL, rAL, sCR, rCR, sCL, rCL):
        my = lax.axis_index("i")
        left = (my + N_DEV - 1) % N_DEV
        right = (my + 1) % N_DEV
        diag = (my + 2) % N_DEV

        barrier = pltpu.get_barrier_semaphore()
        for nbr in (left, right):
            pl.semaphore_signal(
                barrier, inc=1, device_id=(nbr,),
                device_id_type=pl.DeviceIdType.MESH,
            )
        pl.semaphore_wait(barrier, 2)

        def rdma(src, dst, send_sem, recv_sem, dev):
            pltpu.make_async_remote_copy(
                src_ref=src, dst_ref=dst,
                send_sem=send_sem, recv_sem=recv_sem,
                device_id=(dev,), device_id_type=pl.DeviceIdType.MESH,
            ).start()

        def wait_recv(dst, recv_sem):
            pltpu.make_async_remote_copy(
                src_ref=dst, dst_ref=dst,
                send_sem=sAR.at[0], recv_sem=recv_sem,
                device_id=(right,), device_id_type=pl.DeviceIdType.MESH,
            ).wait_recv()

        def wait_send(src, send_sem):
            pltpu.make_async_remote_copy(
                src_ref=src, dst_ref=src,
                send_sem=send_sem, recv_sem=rAR.at[0],
                device_id=(right,), device_id_type=pl.DeviceIdType.MESH,
            ).wait_send()

        def matmul_tile(src_slice, out_rows):
            a_in = pltpu.make_async_copy(src_slice, a_vmem, local_sem)
            a_in.start()
            a_in.wait()
            c_vmem[...] = jnp.dot(
                a_vmem[...], b_ref[...], preferred_element_type=jnp.float32
            )
            c_out = pltpu.make_async_copy(
                c_vmem, out_ref.at[pl.ds(out_rows, TM), :], local_sem
            )
            c_out.start()
            c_out.wait()

        def own_tile(t):
            matmul_tile(a_ref.at[pl.ds(t * TM, TM), :], my * m_per + t * TM)

        def remote_tile(o, j):
            matmul_tile(
                ag_ref.at[o, pl.ds(j * TM, TM), :],
                o * m_per + half + j * TM,
            )

        def a_piece(o, p):
            return ag_ref.at[o, pl.ds(p * piece, piece), :]

        def c_tile(o, t):
            return out_ref.at[pl.ds(o * m_per + t * TM, TM), :]

        for p in range(2):
            src = a_ref.at[pl.ds(half + p * piece, piece), :]
            rdma(src, a_piece(my, p), sAR.at[p], rAR.at[p], right)
            rdma(src, a_piece(my, p), sAL.at[p], rAL.at[p], left)

        def own_c_tile(t, _):
            own_tile(t)
            rdma(c_tile(my, t), c_tile(my, t), sCR.at[t], rCR.at[t], right)
            rdma(c_tile(my, t), c_tile(my, t), sCL.at[t], rCL.at[t], left)
            return _

        lax.fori_loop(0, 2, own_c_tile, 0)

        wait_recv(a_piece(left, 0), rAR.at[0])
        rdma(a_piece(left, 0), a_piece(left, 0), sAR.at[2], rAR.at[2], right)

        lax.fori_loop(2, n_ctiles, own_c_tile, 0)

        wait_recv(a_piece(right, 1), rAL.at[1])
        rdma(a_piece(right, 1), a_piece(right, 1), sAL.at[2], rAL.at[2], left)

        wait_recv(a_piece(left, 1), rAR.at[1])
        lax.fori_loop(0, n_atiles, lambda j, _: (remote_tile(left, j), _)[1], 0)
        wait_recv(a_piece(right, 0), rAL.at[0])
        lax.fori_loop(0, n_atiles, lambda j, _: (remote_tile(right, j), _)[1], 0)

        for j in range(2):
            wait_recv(c_tile(left, j), rCR.at[j])
            rdma(c_tile(left, j), c_tile(left, j),
                 sCR.at[n_ctiles + j], rCR.at[n_ctiles + j], right)
            jr = 2 + j
            wait_recv(c_tile(right, jr), rCL.at[jr])
            rdma(c_tile(right, jr), c_tile(right, jr),
                 sCL.at[n_ctiles + j], rCL.at[n_ctiles + j], left)

        lax.fori_loop(n_ctiles, n_tiles, lambda t, _: (own_tile(t), _)[1], 0)

        wait_recv(a_piece(diag, 0), rAR.at[2])
        wait_recv(a_piece(diag, 1), rAL.at[2])
        lax.fori_loop(0, n_atiles, lambda j, _: (remote_tile(diag, j), _)[1], 0)

        wait_recv(c_tile(left, 2), rCR.at[2])
        wait_recv(c_tile(left, 3), rCR.at[3])
        wait_recv(c_tile(right, 0), rCL.at[0])
        wait_recv(c_tile(right, 1), rCL.at[1])
        wait_recv(c_tile(diag, 0), rCR.at[n_ctiles])
        wait_recv(c_tile(diag, 1), rCR.at[n_ctiles + 1])
        wait_recv(c_tile(diag, 2), rCL.at[n_ctiles])
        wait_recv(c_tile(diag, 3), rCL.at[n_ctiles + 1])

        for p in range(3):
            wait_send(a_piece(my, p % 2), sAR.at[p])
            wait_send(a_piece(my, p % 2), sAL.at[p])
        for t in range(n_ctiles + 2):
            wait_send(c_tile(my, t % n_ctiles), sCR.at[t])
            wait_send(c_tile(my, t % n_ctiles), sCL.at[t])

    return pl.pallas_call(
        body,
        out_shape=[
            jax.ShapeDtypeStruct((M, n), jnp.float32),
            jax.ShapeDtypeStruct((N_DEV, half, k), jnp.float32),
        ],
        in_specs=[
            pl.BlockSpec(memory_space=pl.ANY),
            pl.BlockSpec(memory_space=pltpu.VMEM),
        ],
        out_specs=[
            pl.BlockSpec(memory_space=pl.ANY),
            pl.BlockSpec(memory_space=pl.ANY),
        ],
        scratch_shapes=[
            pltpu.VMEM((TM, k), jnp.float32),
            pltpu.VMEM((TM, n), jnp.float32),
            pltpu.SemaphoreType.DMA,
            pltpu.SemaphoreType.DMA((3,)),
            pltpu.SemaphoreType.DMA((3,)),
            pltpu.SemaphoreType.DMA((3,)),
            pltpu.SemaphoreType.DMA((3,)),
            pltpu.SemaphoreType.DMA((6,)),
            pltpu.SemaphoreType.DMA((6,)),
            pltpu.SemaphoreType.DMA((6,)),
            pltpu.SemaphoreType.DMA((6,)),
        ],
        compiler_params=pltpu.CompilerParams(
            collective_id=0, vmem_limit_bytes=60 * 1024 * 1024
        ),
    )(A, B)[0]
